# baseline (speedup 1.0000x reference)
import jax
import jax.numpy as jnp
from jax.experimental import pallas as pl
from jax.experimental.pallas import tpu as pltpu

EPS = 1e-05


def _block_stats(ref):
    R = ref.shape[1]
    v = ref[0, 0]
    s = v
    q = v * v
    for r in range(1, R):
        v = ref[0, r]
        s = s + v
        q = q + v * v
    return jnp.sum(s, axis=1), jnp.sum(q, axis=1)


def _spe_kernel(s_ref, xs_ref, xn_ref, out_ref):
    C = xs_ref.shape[1] * xs_ref.shape[3]
    R = xs_ref.shape[1]

    sum_n, sumsq_n = _block_stats(xn_ref)
    sum_s, sumsq_s = _block_stats(xs_ref)

    mean_n = sum_n * (1.0 / C)
    var_n = (sumsq_n - sum_n * mean_n) * (1.0 / (C - 1))
    mean_s = sum_s * (1.0 / C)
    var_s = (sumsq_s - sum_s * mean_s) * (1.0 / (C - 1))

    ratio = jnp.sqrt((var_n + EPS) / (var_s + EPS))
    offset = mean_n - mean_s * ratio

    rb = ratio[:, None, :]
    ob = offset[:, None, :]
    for r in range(R):
        out_ref[0, r] = xs_ref[0, r] * rb + ob


def kernel(x, idx_swap):
    N, C, H, W = x.shape
    HW = H * W
    xt = x.reshape(N, C // 8, 8, HW // 128, 128).transpose(0, 1, 3, 2, 4)
    bshape = (1, C // 8, HW // 128, 8, 128)

    grid_spec = pltpu.PrefetchScalarGridSpec(
        num_scalar_prefetch=1,
        grid=(N,),
        in_specs=[
            pl.BlockSpec(bshape, lambda n, s: (s[n], 0, 0, 0, 0)),
            pl.BlockSpec(bshape, lambda n, s: (n, 0, 0, 0, 0)),
        ],
        out_specs=pl.BlockSpec(bshape, lambda n, s: (n, 0, 0, 0, 0)),
    )

    out_t = pl.pallas_call(
        _spe_kernel,
        grid_spec=grid_spec,
        out_shape=jax.ShapeDtypeStruct((N, C // 8, HW // 128, 8, 128), jnp.float32),
    )(idx_swap, xt, xt)
    return out_t.transpose(0, 1, 3, 2, 4).reshape(N, C, H, W)


# speedup vs baseline: 2.5474x; 2.5474x over previous
import jax
import jax.numpy as jnp
from jax.experimental import pallas as pl
from jax.experimental.pallas import tpu as pltpu

EPS = 1e-05

LANE_CHUNK = 1024


def _block_stats(ref, lo):
    C = ref.shape[1]
    w = slice(lo, lo + LANE_CHUNK)
    v = ref[0, 0:8, w]
    s = v
    q = v * v
    for k in range(1, C // 8):
        v = ref[0, 8 * k : 8 * k + 8, w]
        s = s + v
        q = q + v * v
    ssum = jnp.sum(s, axis=0, keepdims=True)
    ssumsq = jnp.sum(q, axis=0, keepdims=True)
    return ssum, ssumsq


def _spe_kernel(ord_ref, src_ref, xs_ref, xn_ref, out_ref):
    C = xn_ref.shape[1]
    HW = xn_ref.shape[2]

    for lo in range(0, HW, LANE_CHUNK):
        w = slice(lo, lo + LANE_CHUNK)
        sum_n, sumsq_n = _block_stats(xn_ref, lo)
        sum_s, sumsq_s = _block_stats(xs_ref, lo)

        mean_n = sum_n * (1.0 / C)
        var_n = (sumsq_n - sum_n * mean_n) * (1.0 / (C - 1))
        mean_s = sum_s * (1.0 / C)
        var_s = (sumsq_s - sum_s * mean_s) * (1.0 / (C - 1))

        ratio = jnp.sqrt((var_n + EPS) / (var_s + EPS))
        offset = mean_n - mean_s * ratio

        for k in range(C // 8):
            sl = slice(8 * k, 8 * k + 8)
            out_ref[0, sl, w] = xs_ref[0, sl, w] * ratio + offset


def kernel(x, idx_swap):
    N, C, H, W = x.shape
    HW = H * W
    xv = x.reshape(N, C, HW)

    order = jnp.argsort(idx_swap).astype(jnp.int32)
    src = jnp.take(idx_swap, order)

    grid_spec = pltpu.PrefetchScalarGridSpec(
        num_scalar_prefetch=2,
        grid=(N,),
        in_specs=[
            pl.BlockSpec((1, C, HW), lambda i, o, s: (s[i], 0, 0)),
            pl.BlockSpec((1, C, HW), lambda i, o, s: (o[i], 0, 0)),
        ],
        out_specs=pl.BlockSpec((1, C, HW), lambda i, o, s: (o[i], 0, 0)),
    )

    out = pl.pallas_call(
        _spe_kernel,
        grid_spec=grid_spec,
        out_shape=jax.ShapeDtypeStruct((N, C, HW), jnp.float32),
    )(order, src, xv, xv)
    return out.reshape(N, C, H, W)
